# Initial kernel scaffold; baseline (speedup 1.0000x reference)
import jax
import jax.numpy as jnp
from jax.experimental import pallas as pl


def kernel(local_coords, idx, latent_codes, W_syn, b_syn):
    raise NotImplementedError("write your pallas kernel here")


# speedup vs baseline: 1.3728x; 1.3728x over previous
import functools

import jax
import jax.numpy as jnp
from jax import lax
from jax.experimental import pallas as pl
from jax.experimental.pallas import tpu as pltpu
from jax.experimental.pallas import tpu_sc as plsc

_BS = 1024
_CIN = 2
_HW = 1024
_ZD = 64
_GD = 2 * _ZD
_NCH = 3


def _sc_gather(idx_half, table2):
    info = plsc.get_sparse_core_info()
    num_cores = info.num_cores
    nw = num_cores * info.num_subcores
    bpw = _BS // nw
    mesh = plsc.VectorSubcoreMesh(core_axis_name="c", subcore_axis_name="s")

    @functools.partial(
        pl.kernel,
        mesh=mesh,
        out_type=jax.ShapeDtypeStruct((_BS, _GD), jnp.float32),
        scratch_types=[
            pltpu.VMEM((bpw,), jnp.int32),
            pltpu.VMEM((bpw, _GD), jnp.float32),
            pltpu.SemaphoreType.DMA,
        ],
    )
    def gather_kernel(idx_hbm, table_hbm, out_hbm, idx_v, rows_v, sem):
        wid = lax.axis_index("s") * num_cores + lax.axis_index("c")
        base = wid * bpw
        pltpu.sync_copy(idx_hbm.at[pl.ds(base, bpw)], idx_v)
        pltpu.async_copy(table_hbm.at[idx_v], rows_v, sem).wait()
        pltpu.sync_copy(rows_v, out_hbm.at[pl.ds(base, bpw)])

    return gather_kernel(idx_half, table2)


def _dense_body(lc_ref, g_ref, par_ref, w01_ref, b_ref, w2lo_ref, w2hi_ref,
                out_ref):
    g = g_ref[...]
    p = par_ref[...]
    v_lo = jnp.dot(g, w2lo_ref[...], preferred_element_type=jnp.float32)
    v_hi = jnp.dot(g, w2hi_ref[...], preferred_element_type=jnp.float32)
    v = v_lo * (1.0 - p) + v_hi * p
    lc0 = lc_ref[:, 0, :]
    lc1 = lc_ref[:, 1, :]
    for ch in range(_NCH):
        out_ref[:, ch, :] = (
            lc0 * w01_ref[0, ch]
            + lc1 * w01_ref[1, ch]
            + (v[:, ch] + b_ref[ch])[:, None]
        )


def _dense(lc, g, parity, w01, b_syn, w2lo, w2hi, block_bs=128):
    grid = (_BS // block_bs,)
    return pl.pallas_call(
        _dense_body,
        grid=grid,
        in_specs=[
            pl.BlockSpec((block_bs, _CIN, _HW), lambda i: (i, 0, 0)),
            pl.BlockSpec((block_bs, _GD), lambda i: (i, 0)),
            pl.BlockSpec((block_bs, 1), lambda i: (i, 0)),
            pl.BlockSpec(memory_space=pltpu.SMEM),
            pl.BlockSpec(memory_space=pltpu.SMEM),
            pl.BlockSpec((_GD, _NCH), lambda i: (0, 0)),
            pl.BlockSpec((_GD, _NCH), lambda i: (0, 0)),
        ],
        out_specs=pl.BlockSpec((block_bs, _NCH, _HW), lambda i: (i, 0, 0)),
        out_shape=jax.ShapeDtypeStruct((_BS, _NCH, _HW), jnp.float32),
    )(lc, g, parity, w01, b_syn, w2lo, w2hi)


def kernel(local_coords, idx, latent_codes, W_syn, b_syn):
    bs, c, h, w = local_coords.shape
    lc = local_coords.reshape(bs, c, h * w)
    idx32 = idx.astype(jnp.int32)
    table2 = latent_codes.reshape(-1, _GD)
    g = _sc_gather(idx32 // 2, table2)
    parity = (idx32 & 1).astype(jnp.float32).reshape(bs, 1)
    w01 = W_syn[:_CIN]
    w2 = W_syn[_CIN:]
    zeros = jnp.zeros_like(w2)
    w2lo = jnp.concatenate([w2, zeros], axis=0)
    w2hi = jnp.concatenate([zeros, w2], axis=0)
    out = _dense(lc, g, parity, w01, b_syn, w2lo, w2hi)
    return out.reshape(bs, _NCH, h, w)
